# Initial kernel scaffold; baseline (speedup 1.0000x reference)
import jax
import jax.numpy as jnp
from jax.experimental import pallas as pl


def kernel(x, expert_indices, expert_weights, w1, w2, w3):
    raise NotImplementedError("write your pallas kernel here")


# speedup vs baseline: 8.4001x; 8.4001x over previous
import functools

import jax
import jax.numpy as jnp
from jax import lax
from jax.experimental import pallas as pl
from jax.experimental.pallas import tpu as pltpu
from jax.experimental.pallas import tpu_sc as plsc

ALIGN = 8
BLK = 128
NC = 2
NS = 16
NW = NC * NS


def _ffn_body(off_ref, nblk_ref, xs_ref, ws_ref, w1_ref, w2_ref, w3_ref, out_ref):
    e = pl.program_id(0)
    start = off_ref[e]
    nblk = nblk_ref[e]
    wa = w1_ref[0]
    wb = w2_ref[0]
    wc = w3_ref[0]

    def body(ib, carry):
        base = pl.multiple_of(start + ib * BLK, ALIGN)
        xb = xs_ref[pl.ds(base, BLK), :]
        g = jnp.dot(xb, wa, preferred_element_type=jnp.float32)
        v = jnp.dot(xb, wb, preferred_element_type=jnp.float32)
        h = (g * jax.nn.sigmoid(g)) * v
        o = jnp.dot(h, wc, preferred_element_type=jnp.float32)
        o = o * ws_ref[pl.ds(base, BLK), :]
        out_ref[pl.ds(base, BLK), :] = o
        return carry

    lax.fori_loop(0, nblk, body, 0)


def _sc_gather_fn(p_tot, d_model):
    bpw = p_tot // NW
    mesh = plsc.VectorSubcoreMesh(
        core_axis_name="c", subcore_axis_name="s", num_cores=NC, num_subcores=NS
    )

    @functools.partial(
        pl.kernel,
        out_type=jax.ShapeDtypeStruct((p_tot, d_model), jnp.float32),
        mesh=mesh,
        scratch_types=[
            pltpu.VMEM((bpw,), jnp.int32),
            pltpu.VMEM((bpw, d_model), jnp.float32),
            pltpu.SemaphoreType.DMA,
        ],
    )
    def gather(x_hbm, idx_hbm, xs_hbm, idx_v, rows_v, sem):
        wid = lax.axis_index("s") * NC + lax.axis_index("c")
        base = wid * bpw
        pltpu.sync_copy(idx_hbm.at[pl.ds(base, bpw)], idx_v)
        pltpu.async_copy(x_hbm.at[idx_v], rows_v, sem).wait()
        pltpu.sync_copy(rows_v, xs_hbm.at[pl.ds(base, bpw)])

    return gather


def _sc_scatter_fn(p_tot, n_out, d_model):
    bpw = p_tot // NW
    mesh = plsc.VectorSubcoreMesh(
        core_axis_name="c", subcore_axis_name="s", num_cores=NC, num_subcores=NS
    )

    @functools.partial(
        pl.kernel,
        out_type=jax.ShapeDtypeStruct((n_out, d_model), jnp.float32),
        mesh=mesh,
        scratch_types=[
            pltpu.VMEM((bpw,), jnp.int32),
            pltpu.VMEM((bpw, d_model), jnp.float32),
            pltpu.SemaphoreType.DMA,
        ],
    )
    def scatter(ys_hbm, tgt_hbm, out_hbm, idx_v, rows_v, sem):
        wid = lax.axis_index("s") * NC + lax.axis_index("c")
        base = wid * bpw
        pltpu.sync_copy(tgt_hbm.at[pl.ds(base, bpw)], idx_v)
        pltpu.sync_copy(ys_hbm.at[pl.ds(base, bpw)], rows_v)
        pltpu.async_copy(rows_v, out_hbm.at[idx_v], sem).wait()

    return scatter


@jax.jit
def kernel(x, expert_indices, expert_weights, w1, w2, w3):
    n_tokens, d_model = x.shape
    n_exp = w1.shape[0]
    d_ff = w1.shape[2]

    p_raw = n_tokens + n_exp * (ALIGN - 1) + BLK
    p_tot = ((p_raw + NW * ALIGN - 1) // (NW * ALIGN)) * (NW * ALIGN)

    e_flat = expert_indices.reshape(-1).astype(jnp.int32)
    w_flat = expert_weights.reshape(-1).astype(jnp.float32)
    order = jnp.argsort(e_flat).astype(jnp.int32)
    counts = jnp.bincount(e_flat, length=n_exp).astype(jnp.int32)
    counts_a = ((counts + ALIGN - 1) // ALIGN) * ALIGN
    zeros1 = jnp.zeros((1,), jnp.int32)
    off_pad = jnp.concatenate([zeros1, jnp.cumsum(counts_a)[:-1].astype(jnp.int32)])
    off_dense = jnp.concatenate([zeros1, jnp.cumsum(counts)[:-1].astype(jnp.int32)])
    sorted_e = e_flat[order]
    rank = jnp.arange(e_flat.shape[0], dtype=jnp.int32) - off_dense[sorted_e]
    pos = off_pad[sorted_e] + rank

    idx_g = jnp.zeros((p_tot,), jnp.int32).at[pos].set(order)
    tgt = jnp.full((p_tot,), n_tokens, jnp.int32).at[pos].set(order)
    ws = jnp.zeros((p_tot, 1), jnp.float32).at[pos, 0].set(w_flat[order])
    nblocks = (counts_a + BLK - 1) // BLK

    xs = _sc_gather_fn(p_tot, d_model)(x, idx_g)

    ys = pl.pallas_call(
        _ffn_body,
        grid=(n_exp,),
        in_specs=[
            pl.BlockSpec(memory_space=pltpu.SMEM),
            pl.BlockSpec(memory_space=pltpu.SMEM),
            pl.BlockSpec((p_tot, d_model), lambda e: (0, 0)),
            pl.BlockSpec((p_tot, 1), lambda e: (0, 0)),
            pl.BlockSpec((1, d_model, d_ff), lambda e: (e, 0, 0)),
            pl.BlockSpec((1, d_model, d_ff), lambda e: (e, 0, 0)),
            pl.BlockSpec((1, d_ff, d_model), lambda e: (e, 0, 0)),
        ],
        out_specs=pl.BlockSpec((p_tot, d_model), lambda e: (0, 0)),
        out_shape=jax.ShapeDtypeStruct((p_tot, d_model), jnp.float32),
        compiler_params=pltpu.CompilerParams(
            dimension_semantics=("arbitrary",),
        ),
    )(off_pad, nblocks, xs, ws, w1, w2, w3)

    n_out = n_tokens + ALIGN
    out_ext = _sc_scatter_fn(p_tot, n_out, d_model)(ys, tgt)
    return out_ext[:n_tokens]


# speedup vs baseline: 9.2490x; 1.1011x over previous
import functools

import jax
import jax.numpy as jnp
from jax import lax
from jax.experimental import pallas as pl
from jax.experimental.pallas import tpu as pltpu
from jax.experimental.pallas import tpu_sc as plsc

ALIGN = 8
BLK = 128
NC = 2
NS = 16
NW = NC * NS


def _ffn_body(off_ref, nblk_ref, xs_ref, ws_ref, w1_ref, w2_ref, w3_ref, out_ref):
    e = pl.program_id(0)
    start = off_ref[e]
    nblk = nblk_ref[e]
    wa = w1_ref[0]
    wb = w2_ref[0]
    wc = w3_ref[0]

    def body(ib, carry):
        base = pl.multiple_of(start + ib * BLK, ALIGN)
        xb = xs_ref[pl.ds(base, BLK), :]
        g = jnp.dot(xb, wa, preferred_element_type=jnp.float32)
        v = jnp.dot(xb, wb, preferred_element_type=jnp.float32)
        h = (g * jax.nn.sigmoid(g)) * v
        o = jnp.dot(h, wc, preferred_element_type=jnp.float32)
        o = o * ws_ref[pl.ds(base, BLK), :]
        out_ref[pl.ds(base, BLK), :] = o
        return carry

    lax.fori_loop(0, nblk, body, 0)


def _sc_gather_fn(n_idx, d_model):
    bpw = n_idx // NW
    mesh = plsc.VectorSubcoreMesh(
        core_axis_name="c", subcore_axis_name="s", num_cores=NC, num_subcores=NS
    )

    @functools.partial(
        pl.kernel,
        out_type=jax.ShapeDtypeStruct((n_idx, d_model), jnp.float32),
        mesh=mesh,
        scratch_types=[
            pltpu.VMEM((bpw,), jnp.int32),
            pltpu.VMEM((bpw, d_model), jnp.float32),
            pltpu.SemaphoreType.DMA,
        ],
    )
    def gather(table_hbm, idx_hbm, out_hbm, idx_v, rows_v, sem):
        wid = lax.axis_index("s") * NC + lax.axis_index("c")
        base = wid * bpw
        pltpu.sync_copy(idx_hbm.at[pl.ds(base, bpw)], idx_v)
        pltpu.async_copy(table_hbm.at[idx_v], rows_v, sem).wait()
        pltpu.sync_copy(rows_v, out_hbm.at[pl.ds(base, bpw)])

    return gather


@jax.jit
def kernel(x, expert_indices, expert_weights, w1, w2, w3):
    n_tokens, d_model = x.shape
    n_exp = w1.shape[0]
    d_ff = w1.shape[2]

    p_raw = n_tokens + n_exp * (ALIGN - 1) + BLK
    p_tot = ((p_raw + NW * ALIGN - 1) // (NW * ALIGN)) * (NW * ALIGN)

    e_flat = expert_indices.reshape(-1).astype(jnp.int32)
    w_flat = expert_weights.reshape(-1).astype(jnp.float32)
    order = jnp.argsort(e_flat).astype(jnp.int32)
    counts = jnp.bincount(e_flat, length=n_exp).astype(jnp.int32)
    counts_a = ((counts + ALIGN - 1) // ALIGN) * ALIGN
    zeros1 = jnp.zeros((1,), jnp.int32)
    off_pad = jnp.concatenate([zeros1, jnp.cumsum(counts_a)[:-1].astype(jnp.int32)])
    off_dense = jnp.concatenate([zeros1, jnp.cumsum(counts)[:-1].astype(jnp.int32)])
    sorted_e = e_flat[order]
    rank = jnp.arange(e_flat.shape[0], dtype=jnp.int32) - off_dense[sorted_e]
    pos = off_pad[sorted_e] + rank

    idx_g = jnp.zeros((p_tot,), jnp.int32).at[pos].set(order)
    inv_pos = jnp.zeros((n_tokens,), jnp.int32).at[order].set(pos)
    ws = jnp.zeros((p_tot, 1), jnp.float32).at[pos, 0].set(w_flat[order])
    nblocks = (counts_a + BLK - 1) // BLK

    xs = _sc_gather_fn(p_tot, d_model)(x, idx_g)

    ys = pl.pallas_call(
        _ffn_body,
        grid=(n_exp,),
        in_specs=[
            pl.BlockSpec(memory_space=pltpu.SMEM),
            pl.BlockSpec(memory_space=pltpu.SMEM),
            pl.BlockSpec((p_tot, d_model), lambda e: (0, 0)),
            pl.BlockSpec((p_tot, 1), lambda e: (0, 0)),
            pl.BlockSpec((1, d_model, d_ff), lambda e: (e, 0, 0)),
            pl.BlockSpec((1, d_model, d_ff), lambda e: (e, 0, 0)),
            pl.BlockSpec((1, d_ff, d_model), lambda e: (e, 0, 0)),
        ],
        out_specs=pl.BlockSpec((p_tot, d_model), lambda e: (0, 0)),
        out_shape=jax.ShapeDtypeStruct((p_tot, d_model), jnp.float32),
        compiler_params=pltpu.CompilerParams(
            dimension_semantics=("arbitrary",),
        ),
    )(off_pad, nblocks, xs, ws, w1, w2, w3)

    return _sc_gather_fn(n_tokens, d_model)(ys, inv_pos)


# speedup vs baseline: 16.2093x; 1.7526x over previous
import functools

import jax
import jax.numpy as jnp
from jax import lax
from jax.experimental import pallas as pl
from jax.experimental.pallas import tpu as pltpu
from jax.experimental.pallas import tpu_sc as plsc

ALIGN = 8
BLK = 128
MBLK = 128
NC = 2
NS = 16
NW = NC * NS
WLANES = 128


def _meta_body(e_ref, inv_ref, off_ref, nblk_ref):
    n_tok = e_ref.shape[0]
    n_exp = off_ref.shape[1]
    ecol = e_ref[:]
    onehot = (
        ecol == lax.broadcasted_iota(jnp.int32, (n_tok, n_exp), 1)
    ).astype(jnp.float32)

    counts = jnp.sum(onehot, axis=0, keepdims=True).astype(jnp.int32)
    counts_a = ((counts + ALIGN - 1) // ALIGN) * ALIGN
    lt_e = (
        lax.broadcasted_iota(jnp.int32, (n_exp, n_exp), 0)
        < lax.broadcasted_iota(jnp.int32, (n_exp, n_exp), 1)
    ).astype(jnp.float32)
    off_f = jnp.dot(
        counts_a.astype(jnp.float32), lt_e, preferred_element_type=jnp.float32
    )

    tri = (
        lax.broadcasted_iota(jnp.int32, (MBLK, MBLK), 0)
        > lax.broadcasted_iota(jnp.int32, (MBLK, MBLK), 1)
    ).astype(jnp.float32)

    carry = jnp.zeros((1, n_exp), jnp.float32)
    for b in range(n_tok // MBLK):
        ob = onehot[b * MBLK : (b + 1) * MBLK, :]
        cb = jnp.dot(tri, ob, preferred_element_type=jnp.float32) + carry
        rank_b = jnp.sum(cb * ob, axis=1, keepdims=True)
        base_b = jnp.sum(off_f * ob, axis=1, keepdims=True)
        inv_ref[b * MBLK : (b + 1) * MBLK, :] = (rank_b + base_b).astype(jnp.int32)
        carry = carry + jnp.sum(ob, axis=0, keepdims=True)

    off_ref[...] = off_f.astype(jnp.int32)
    nblk_ref[...] = (counts_a + BLK - 1) // BLK


def _ffn_body(off_ref, nblk_ref, xs_ref, ws_ref, w1_ref, w2_ref, w3_ref, out_ref):
    e = pl.program_id(0)
    start = off_ref[0, e]
    nblk = nblk_ref[0, e]
    wa = w1_ref[0]
    wb = w2_ref[0]
    wc = w3_ref[0]

    def body(ib, carry):
        base = pl.multiple_of(start + ib * BLK, ALIGN)
        xb = xs_ref[pl.ds(base, BLK), :]
        g = jnp.dot(xb, wa, preferred_element_type=jnp.float32)
        v = jnp.dot(xb, wb, preferred_element_type=jnp.float32)
        h = (g * jax.nn.sigmoid(g)) * v
        o = jnp.dot(h, wc, preferred_element_type=jnp.float32)
        o = o * ws_ref[pl.ds(base, BLK), :][:, 0:1]
        out_ref[pl.ds(base, BLK), :] = o
        return carry

    lax.fori_loop(0, nblk, body, 0)


def _sc_mesh():
    return plsc.VectorSubcoreMesh(
        core_axis_name="c", subcore_axis_name="s", num_cores=NC, num_subcores=NS
    )


def _sc_dispatch_fn(n_tokens, p_tot, d_model):
    bpw = n_tokens // NW

    @functools.partial(
        pl.kernel,
        out_type=(
            jax.ShapeDtypeStruct((p_tot, d_model), jnp.float32),
            jax.ShapeDtypeStruct((p_tot, WLANES), jnp.float32),
        ),
        mesh=_sc_mesh(),
        scratch_types=[
            pltpu.VMEM((bpw,), jnp.int32),
            pltpu.VMEM((bpw, d_model), jnp.float32),
            pltpu.VMEM((bpw, WLANES), jnp.float32),
            pltpu.SemaphoreType.DMA,
            pltpu.SemaphoreType.DMA,
        ],
    )
    def dispatch(x_hbm, wrow_hbm, inv_hbm, xs_hbm, ws_hbm, idx_v, rows_v, wrows_v, sem_a, sem_b):
        wid = lax.axis_index("s") * NC + lax.axis_index("c")
        base = wid * bpw
        pltpu.sync_copy(inv_hbm.at[pl.ds(base, bpw)], idx_v)
        pltpu.sync_copy(x_hbm.at[pl.ds(base, bpw)], rows_v)
        pltpu.sync_copy(wrow_hbm.at[pl.ds(base, bpw)], wrows_v)
        cp_a = pltpu.async_copy(rows_v, xs_hbm.at[idx_v], sem_a)
        cp_b = pltpu.async_copy(wrows_v, ws_hbm.at[idx_v], sem_b)
        cp_a.wait()
        cp_b.wait()

    return dispatch


def _sc_combine_fn(n_tokens, p_tot, d_model):
    bpw = n_tokens // NW

    @functools.partial(
        pl.kernel,
        out_type=jax.ShapeDtypeStruct((n_tokens, d_model), jnp.float32),
        mesh=_sc_mesh(),
        scratch_types=[
            pltpu.VMEM((bpw,), jnp.int32),
            pltpu.VMEM((bpw, d_model), jnp.float32),
            pltpu.SemaphoreType.DMA,
        ],
    )
    def combine(ys_hbm, inv_hbm, out_hbm, idx_v, rows_v, sem):
        wid = lax.axis_index("s") * NC + lax.axis_index("c")
        base = wid * bpw
        pltpu.sync_copy(inv_hbm.at[pl.ds(base, bpw)], idx_v)
        pltpu.async_copy(ys_hbm.at[idx_v], rows_v, sem).wait()
        pltpu.sync_copy(rows_v, out_hbm.at[pl.ds(base, bpw)])

    return combine


@jax.jit
def kernel(x, expert_indices, expert_weights, w1, w2, w3):
    n_tokens, d_model = x.shape
    n_exp = w1.shape[0]
    d_ff = w1.shape[2]

    p_tot = n_tokens + n_exp * (ALIGN - 1) + BLK
    p_tot = ((p_tot + BLK - 1) // BLK) * BLK

    e_col = expert_indices.reshape(n_tokens, 1).astype(jnp.int32)
    wrow = jnp.broadcast_to(
        expert_weights.reshape(n_tokens, 1).astype(jnp.float32), (n_tokens, WLANES)
    )

    inv2d, off2d, nblk2d = pl.pallas_call(
        _meta_body,
        out_shape=(
            jax.ShapeDtypeStruct((n_tokens, 1), jnp.int32),
            jax.ShapeDtypeStruct((1, n_exp), jnp.int32),
            jax.ShapeDtypeStruct((1, n_exp), jnp.int32),
        ),
    )(e_col)
    inv = inv2d.reshape(n_tokens)

    xs, ws = _sc_dispatch_fn(n_tokens, p_tot, d_model)(x, wrow, inv)

    ys = pl.pallas_call(
        _ffn_body,
        grid=(n_exp,),
        in_specs=[
            pl.BlockSpec(memory_space=pltpu.SMEM),
            pl.BlockSpec(memory_space=pltpu.SMEM),
            pl.BlockSpec((p_tot, d_model), lambda e: (0, 0)),
            pl.BlockSpec((p_tot, WLANES), lambda e: (0, 0)),
            pl.BlockSpec((1, d_model, d_ff), lambda e: (e, 0, 0)),
            pl.BlockSpec((1, d_model, d_ff), lambda e: (e, 0, 0)),
            pl.BlockSpec((1, d_ff, d_model), lambda e: (e, 0, 0)),
        ],
        out_specs=pl.BlockSpec((p_tot, d_model), lambda e: (0, 0)),
        out_shape=jax.ShapeDtypeStruct((p_tot, d_model), jnp.float32),
        compiler_params=pltpu.CompilerParams(
            dimension_semantics=("arbitrary",),
        ),
    )(off2d, nblk2d, xs, ws, w1, w2, w3)

    return _sc_combine_fn(n_tokens, p_tot, d_model)(ys, inv)
